# baseline (speedup 1.0000x reference)
import functools

import jax
import jax.numpy as jnp
from jax import lax
from jax.experimental import pallas as pl
from jax.experimental.pallas import tpu as pltpu
from jax.experimental.pallas import tpu_sc as plsc

_E = 512
_BS = 128
_N = 63
_PADN = 64
_ROWS = _PADN * _BS

_NC, _NS = 2, 16
_NW = _NC * _NS
_BPW = _ROWS // _NW
_CH = 4
_CROWS = _BPW // _CH


@functools.cache
def _make_sc_gather():
    mesh = plsc.VectorSubcoreMesh(core_axis_name="c", subcore_axis_name="s")

    @functools.partial(
        pl.kernel,
        mesh=mesh,
        out_type=jax.ShapeDtypeStruct((_N * _BS, _E), jnp.float32),
        scratch_types=[
            pltpu.VMEM((_BPW,), jnp.int32),
            pltpu.VMEM((_CROWS, _E), jnp.float32),
            pltpu.VMEM((_CROWS, _E), jnp.float32),
            pltpu.VMEM((_CROWS, _E), jnp.float32),
            pltpu.SemaphoreType.DMA,
            pltpu.SemaphoreType.DMA,
            pltpu.SemaphoreType.DMA,
            pltpu.SemaphoreType.DMA,
            pltpu.SemaphoreType.DMA,
            pltpu.SemaphoreType.DMA,
        ],
    )
    def _sc_gather(emb_hbm, idx_hbm, out_hbm, idx_v, buf0, buf1, buf2,
                   gs0, gs1, gs2, os0, os1, os2):
        wid = lax.axis_index("s") * _NC + lax.axis_index("c")
        base = wid * _BPW
        bufs = (buf0, buf1, buf2)
        gsems = (gs0, gs1, gs2)
        osems = (os0, os1, os2)

        def pipeline(nch):
            gcp = [None] * nch
            ocp = [None] * nch
            for c in range(min(3, nch)):
                gcp[c] = pltpu.async_copy(
                    emb_hbm.at[idx_v.at[pl.ds(c * _CROWS, _CROWS)]],
                    bufs[c % 3], gsems[c % 3])
            for c in range(nch):
                gcp[c].wait()
                ocp[c] = pltpu.async_copy(
                    bufs[c % 3],
                    out_hbm.at[pl.ds(base + c * _CROWS, _CROWS)],
                    osems[c % 3])
                if c + 3 < nch:
                    ocp[c].wait()
                    gcp[c + 3] = pltpu.async_copy(
                        emb_hbm.at[idx_v.at[pl.ds((c + 3) * _CROWS,
                                                  _CROWS)]],
                        bufs[c % 3], gsems[c % 3])
            for c in range(max(0, nch - 3), nch):
                ocp[c].wait()

        @pl.when(wid < _NW - 1)
        def _full():
            pltpu.sync_copy(idx_hbm.at[pl.ds(base, _BPW)], idx_v)
            pipeline(_CH)

        @pl.when(wid == _NW - 1)
        def _tail():
            pltpu.sync_copy(idx_hbm.at[pl.ds(base, _BPW // 2)],
                            idx_v.at[pl.ds(0, _BPW // 2)])
            pipeline(_CH // 2)

    return _sc_gather


def _dot_t(a, b_t):
    return lax.dot_general(a, b_t, (((1,), (1,)), ((), ())),
                           preferred_element_type=jnp.float32)


def _dot(a, b):
    return lax.dot_general(a, b, (((1,), (0,)), ((), ())),
                           preferred_element_type=jnp.float32)


def _gru_combine(gi, gh, h0):
    r = jax.nn.sigmoid(gi[:, :_E] + gh[:, :_E])
    z = jax.nn.sigmoid(gi[:, _E:2 * _E] + gh[:, _E:2 * _E])
    n = jnp.tanh(gi[:, 2 * _E:] + r * gh[:, 2 * _E:])
    return (1.0 - z) * n + z * h0 if h0 is not None else (1.0 - z) * n


def _tree_body(x_hbm, wih_ref, whh_ref, bih_ref, bhh_ref, sw_ref, sb_ref,
               cw_ref, out_ref, h_all, x_vmem, xsems, isem):
    xcp = []
    for k in range(4):
        r = 3968 + k * 1024
        xcp.append(pltpu.make_async_copy(
            x_hbm.at[pl.ds(r, 1024), :], x_vmem.at[pl.ds(r, 1024), :],
            xsems.at[k]))
        xcp[k].start()
    icp = pltpu.make_async_copy(
        x_hbm.at[pl.ds(0, 3968), :], x_vmem.at[pl.ds(0, 3968), :], isem)
    icp.start()

    bih = bih_ref[:]
    bhh = bhh_ref[:]
    wih_b = wih_ref[:, :]
    whh_b = whh_ref[:, :]
    sw_b = sw_ref[:, :]
    cw_b = cw_ref[:, :]

    def gi_at(prow, m):
        x = x_vmem[pl.ds(prow, m), :]
        return _dot_t(x, wih_b) + bih

    def leaf_step(prow, nn):
        m = nn * _BS
        gh = jnp.broadcast_to(bhh, (m, 3 * _E))
        h = _gru_combine(gi_at(prow, m), gh, None)
        h_all[pl.ds(prow, m), :] = h
        return h

    def internal_step(prow, crow, nn):
        m = nn * _BS
        hc = h_all[pl.ds(crow, 2 * m), :]
        u = jnp.tanh(_dot(hc, sw_b) + sb_ref[:, :])
        s = jnp.tanh(_dot(u, cw_b))
        s4 = s.reshape(nn, 2, _BS, 1)
        w1 = jax.nn.sigmoid(s4[:, 0] - s4[:, 1])
        hc4 = hc.reshape(nn, 2, _BS, _E)
        h0 = (w1 * hc4[:, 0] + (1.0 - w1) * hc4[:, 1]).reshape(m, _E)
        gh = _dot_t(h0, whh_b) + bhh
        h = _gru_combine(gi_at(prow, m), gh, h0)
        h_all[pl.ds(prow, m), :] = h
        return h

    def fold(acc, h, nn):
        m = jnp.max(h.reshape(nn, _BS, _E), axis=0) if nn > 1 else h
        return m if acc is None else jnp.maximum(acc, m)

    acc = None
    for k in range(4):
        xcp[k].wait()
        acc = fold(acc, leaf_step(3968 + k * 1024, 8), 8)
    icp.wait()
    for half in range(2):
        acc = fold(acc, internal_step(1920 + half * 1024,
                                      3968 + half * 2048, 8), 8)
    acc = fold(acc, internal_step(896, 1920, 8), 8)
    acc = fold(acc, internal_step(384, 896, 4), 4)
    acc = fold(acc, internal_step(128, 384, 2), 2)
    acc = fold(acc, internal_step(0, 128, 1), 1)
    out_ref[:, :] = acc


def _tree_gru(x, wih, whh, bih, bhh, sw, sb, cw):
    return pl.pallas_call(
        _tree_body,
        in_specs=[pl.BlockSpec(memory_space=pltpu.HBM)]
        + [pl.BlockSpec(memory_space=pltpu.VMEM)] * 7,
        out_specs=pl.BlockSpec(memory_space=pltpu.VMEM),
        out_shape=jax.ShapeDtypeStruct((_BS, _E), jnp.float32),
        scratch_shapes=[
            pltpu.VMEM((_N * _BS, _E), jnp.float32),
            pltpu.VMEM((_N * _BS, _E), jnp.float32),
            pltpu.SemaphoreType.DMA((4,)),
            pltpu.SemaphoreType.DMA,
        ],
    )(x, wih, whh, bih, bhh, sw, sb, cw)


def kernel(node_ids, emb, Wih, Whh, bih, bhh, sent_w, sent_b, ctx_w):
    x = _make_sc_gather()(emb, node_ids.reshape(-1))
    return _tree_gru(x, Wih, Whh, bih, bhh, sent_w, sent_b, ctx_w)


# speedup vs baseline: 1.0222x; 1.0222x over previous
import functools

import jax
import jax.numpy as jnp
from jax import lax
from jax.experimental import pallas as pl
from jax.experimental.pallas import tpu as pltpu
from jax.experimental.pallas import tpu_sc as plsc

_E = 512
_BS = 128
_N = 63
_PADN = 64
_ROWS = _PADN * _BS

_NC, _NS = 2, 16
_NW = _NC * _NS
_BPW = _ROWS // _NW
_CH = 4
_CROWS = _BPW // _CH


@functools.cache
def _make_sc_gather():
    mesh = plsc.VectorSubcoreMesh(core_axis_name="c", subcore_axis_name="s")

    @functools.partial(
        pl.kernel,
        mesh=mesh,
        out_type=jax.ShapeDtypeStruct((_N * _BS, _E), jnp.float32),
        scratch_types=[
            pltpu.VMEM((8, _BS), jnp.int32),
            pltpu.VMEM((_CROWS, _E), jnp.float32),
            pltpu.VMEM((_CROWS, _E), jnp.float32),
            pltpu.VMEM((_CROWS, _E), jnp.float32),
            pltpu.SemaphoreType.DMA,
            pltpu.SemaphoreType.DMA,
            pltpu.SemaphoreType.DMA,
            pltpu.SemaphoreType.DMA,
            pltpu.SemaphoreType.DMA,
            pltpu.SemaphoreType.DMA,
        ],
    )
    def _sc_gather(emb_hbm, idx_hbm, out_hbm, idx_v, buf0, buf1, buf2,
                   gs0, gs1, gs2, os0, os1, os2):
        wid = lax.axis_index("s") * _NC + lax.axis_index("c")
        base = wid * _BPW
        lr0 = 2 * lax.rem(wid, 4)
        bufs = (buf0, buf1, buf2)
        gsems = (gs0, gs1, gs2)
        osems = (os0, os1, os2)

        def pipeline(nch):
            def idx_slice(c):
                return idx_v.at[lr0 + c // 2, pl.ds((c % 2) * _CROWS,
                                                    _CROWS)]

            gcp = [None] * nch
            ocp = [None] * nch
            for c in range(min(3, nch)):
                gcp[c] = pltpu.async_copy(
                    emb_hbm.at[idx_slice(c)], bufs[c % 3], gsems[c % 3])
            for c in range(nch):
                gcp[c].wait()
                ocp[c] = pltpu.async_copy(
                    bufs[c % 3],
                    out_hbm.at[pl.ds(base + c * _CROWS, _CROWS)],
                    osems[c % 3])
                if c + 3 < nch:
                    ocp[c].wait()
                    gcp[c + 3] = pltpu.async_copy(
                        emb_hbm.at[idx_slice(c + 3)],
                        bufs[c % 3], gsems[c % 3])
            for c in range(max(0, nch - 3), nch):
                ocp[c].wait()

        @pl.when(wid < 28)
        def _full():
            pltpu.sync_copy(idx_hbm.at[pl.ds(8 * (wid // 4), 8)], idx_v)
            pipeline(_CH)

        @pl.when((wid >= 28) & (wid < _NW - 1))
        def _near_tail():
            pltpu.sync_copy(idx_hbm.at[pl.ds(56, 7)],
                            idx_v.at[pl.ds(0, 7)])
            pipeline(_CH)

        @pl.when(wid == _NW - 1)
        def _tail():
            pltpu.sync_copy(idx_hbm.at[pl.ds(56, 7)],
                            idx_v.at[pl.ds(0, 7)])
            pipeline(_CH // 2)

    return _sc_gather


def _dot_t(a, b_t):
    return lax.dot_general(a, b_t, (((1,), (1,)), ((), ())),
                           preferred_element_type=jnp.float32)


def _dot(a, b):
    return lax.dot_general(a, b, (((1,), (0,)), ((), ())),
                           preferred_element_type=jnp.float32)


def _gru_combine(gi, gh, h0):
    r = jax.nn.sigmoid(gi[:, :_E] + gh[:, :_E])
    z = jax.nn.sigmoid(gi[:, _E:2 * _E] + gh[:, _E:2 * _E])
    n = jnp.tanh(gi[:, 2 * _E:] + r * gh[:, 2 * _E:])
    return (1.0 - z) * n + z * h0 if h0 is not None else (1.0 - z) * n


def _tree_body(x_hbm, wih_ref, whh_ref, bih_ref, bhh_ref, sw_ref, sb_ref,
               cw_ref, out_ref, h_all, x_vmem, xsems, isem):
    xcp = []
    for k in range(4):
        r = 3968 + k * 1024
        xcp.append(pltpu.make_async_copy(
            x_hbm.at[pl.ds(r, 1024), :], x_vmem.at[pl.ds(r, 1024), :],
            xsems.at[k]))
        xcp[k].start()
    icp = pltpu.make_async_copy(
        x_hbm.at[pl.ds(0, 3968), :], x_vmem.at[pl.ds(0, 3968), :], isem)
    icp.start()

    bih = bih_ref[:]
    bhh = bhh_ref[:]
    wih_b = wih_ref[:, :]
    whh_b = whh_ref[:, :]
    sw_b = sw_ref[:, :]
    cw_b = cw_ref[:, :]

    def gi_at(prow, m):
        x = x_vmem[pl.ds(prow, m), :]
        return _dot_t(x, wih_b) + bih

    def leaf_step(prow, nn):
        m = nn * _BS
        gi = gi_at(prow, m)
        gh = jnp.broadcast_to(bhh, (m, 3 * _E))
        h = _gru_combine(gi, gh, None)
        h_all[pl.ds(prow, m), :] = h
        return h

    def internal_step(prow, crow, nn):
        m = nn * _BS
        hc = h_all[pl.ds(crow, 2 * m), :]
        u = jnp.tanh(_dot(hc, sw_b) + sb_ref[:, :])
        s = jnp.tanh(_dot(u, cw_b))
        s4 = s.reshape(nn, 2, _BS, 1)
        w1 = jax.nn.sigmoid(s4[:, 0] - s4[:, 1])
        hc4 = hc.reshape(nn, 2, _BS, _E)
        h0 = (w1 * hc4[:, 0] + (1.0 - w1) * hc4[:, 1]).reshape(m, _E)
        gi = gi_at(prow, m)
        gh = _dot_t(h0, whh_b) + bhh
        h = _gru_combine(gi, gh, h0)
        h_all[pl.ds(prow, m), :] = h
        return h

    def fold(acc, h, nn):
        m = jnp.max(h.reshape(nn, _BS, _E), axis=0) if nn > 1 else h
        return m if acc is None else jnp.maximum(acc, m)

    acc = None
    for k in range(4):
        xcp[k].wait()
        acc = fold(acc, leaf_step(3968 + k * 1024, 8), 8)
    icp.wait()
    for half in range(2):
        acc = fold(acc, internal_step(1920 + half * 1024,
                                      3968 + half * 2048, 8), 8)
    acc = fold(acc, internal_step(896, 1920, 8), 8)
    acc = fold(acc, internal_step(384, 896, 4), 4)
    acc = fold(acc, internal_step(128, 384, 2), 2)
    acc = fold(acc, internal_step(0, 128, 1), 1)
    out_ref[:, :] = acc


def _tree_gru(x, wih, whh, bih, bhh, sw, sb, cw):
    return pl.pallas_call(
        _tree_body,
        in_specs=[pl.BlockSpec(memory_space=pltpu.HBM)]
        + [pl.BlockSpec(memory_space=pltpu.VMEM)] * 7,
        out_specs=pl.BlockSpec(memory_space=pltpu.VMEM),
        out_shape=jax.ShapeDtypeStruct((_BS, _E), jnp.float32),
        scratch_shapes=[
            pltpu.VMEM((_N * _BS, _E), jnp.float32),
            pltpu.VMEM((_N * _BS, _E), jnp.float32),
            pltpu.SemaphoreType.DMA((4,)),
            pltpu.SemaphoreType.DMA,
        ],
    )(x, wih, whh, bih, bhh, sw, sb, cw)


def kernel(node_ids, emb, Wih, Whh, bih, bhh, sent_w, sent_b, ctx_w):
    x = _make_sc_gather()(emb, node_ids)
    return _tree_gru(x, Wih, Whh, bih, bhh, sent_w, sent_b, ctx_w)


# speedup vs baseline: 1.0268x; 1.0045x over previous
import functools

import jax
import jax.numpy as jnp
from jax import lax
from jax.experimental import pallas as pl
from jax.experimental.pallas import tpu as pltpu
from jax.experimental.pallas import tpu_sc as plsc

_E = 512
_BS = 128
_N = 63
_PADN = 64
_ROWS = _PADN * _BS

_NC, _NS = 2, 16
_NW = _NC * _NS
_BPW = _ROWS // _NW
_CH = 4
_CROWS = _BPW // _CH


@functools.cache
def _make_sc_gather():
    mesh = plsc.VectorSubcoreMesh(core_axis_name="c", subcore_axis_name="s")

    @functools.partial(
        pl.kernel,
        mesh=mesh,
        out_type=jax.ShapeDtypeStruct((_N * _BS, _E), jnp.float32),
        scratch_types=[
            pltpu.VMEM((8, _BS), jnp.int32),
            pltpu.VMEM((_CROWS, _E), jnp.float32),
            pltpu.VMEM((_CROWS, _E), jnp.float32),
            pltpu.VMEM((_CROWS, _E), jnp.float32),
            pltpu.SemaphoreType.DMA,
            pltpu.SemaphoreType.DMA,
            pltpu.SemaphoreType.DMA,
            pltpu.SemaphoreType.DMA,
            pltpu.SemaphoreType.DMA,
            pltpu.SemaphoreType.DMA,
        ],
    )
    def _sc_gather(emb_hbm, idx_hbm, out_hbm, idx_v, buf0, buf1, buf2,
                   gs0, gs1, gs2, os0, os1, os2):
        wid = lax.axis_index("s") * _NC + lax.axis_index("c")
        base = wid * _BPW
        lr0 = 2 * lax.rem(wid, 4)
        bufs = (buf0, buf1, buf2)
        gsems = (gs0, gs1, gs2)
        osems = (os0, os1, os2)

        def pipeline(nch):
            def idx_slice(c):
                return idx_v.at[lr0 + c // 2, pl.ds((c % 2) * _CROWS,
                                                    _CROWS)]

            gcp = [None] * nch
            ocp = [None] * nch
            for c in range(min(3, nch)):
                gcp[c] = pltpu.async_copy(
                    emb_hbm.at[idx_slice(c)], bufs[c % 3], gsems[c % 3])
            for c in range(nch):
                gcp[c].wait()
                ocp[c] = pltpu.async_copy(
                    bufs[c % 3],
                    out_hbm.at[pl.ds(base + c * _CROWS, _CROWS)],
                    osems[c % 3])
                if c + 3 < nch:
                    ocp[c].wait()
                    gcp[c + 3] = pltpu.async_copy(
                        emb_hbm.at[idx_slice(c + 3)],
                        bufs[c % 3], gsems[c % 3])
            for c in range(max(0, nch - 3), nch):
                ocp[c].wait()

        @pl.when(wid < 28)
        def _full():
            pltpu.sync_copy(idx_hbm.at[pl.ds(8 * (wid // 4), 8)], idx_v)
            pipeline(_CH)

        @pl.when((wid >= 28) & (wid < _NW - 1))
        def _near_tail():
            pltpu.sync_copy(idx_hbm.at[pl.ds(56, 7)],
                            idx_v.at[pl.ds(0, 7)])
            pipeline(_CH)

        @pl.when(wid == _NW - 1)
        def _tail():
            pltpu.sync_copy(idx_hbm.at[pl.ds(56, 7)],
                            idx_v.at[pl.ds(0, 7)])
            pipeline(_CH // 2)

    return _sc_gather


def _dot_t(a, b_t):
    return lax.dot_general(a, b_t, (((1,), (1,)), ((), ())),
                           preferred_element_type=jnp.float32)


def _dot(a, b):
    return lax.dot_general(a, b, (((1,), (0,)), ((), ())),
                           preferred_element_type=jnp.float32)


def _gru_combine(gi, gh, h0):
    r = jax.nn.sigmoid(gi[:, :_E] + gh[:, :_E])
    z = jax.nn.sigmoid(gi[:, _E:2 * _E] + gh[:, _E:2 * _E])
    n = jnp.tanh(gi[:, 2 * _E:] + r * gh[:, 2 * _E:])
    return (1.0 - z) * n + z * h0 if h0 is not None else (1.0 - z) * n


def _tree_body(x_hbm, wih_ref, whh_hbm, bih_ref, bhh_ref, sw_hbm, sb_ref,
               cw_ref, out_ref, h_all, x_vmem, whh_v, sw_v, xsems, isem,
               wsem, ssem):
    xcp = []
    for k in range(4):
        r = 3968 + k * 1024
        xcp.append(pltpu.make_async_copy(
            x_hbm.at[pl.ds(r, 1024), :], x_vmem.at[pl.ds(r, 1024), :],
            xsems.at[k]))
        xcp[k].start()
    icp = pltpu.make_async_copy(
        x_hbm.at[pl.ds(0, 3968), :], x_vmem.at[pl.ds(0, 3968), :], isem)
    icp.start()
    wcp = pltpu.make_async_copy(whh_hbm, whh_v, wsem)
    wcp.start()
    scp = pltpu.make_async_copy(sw_hbm, sw_v, ssem)
    scp.start()

    bih = bih_ref[:]
    bhh = bhh_ref[:]
    wih_b = wih_ref[:, :]
    cw_b = cw_ref[:, :]

    def gi_at(prow, m):
        x = x_vmem[pl.ds(prow, m), :]
        return _dot_t(x, wih_b) + bih

    def leaf_step(prow, nn):
        m = nn * _BS
        gi = gi_at(prow, m)
        gh = jnp.broadcast_to(bhh, (m, 3 * _E))
        h = _gru_combine(gi, gh, None)
        h_all[pl.ds(prow, m), :] = h
        return h

    def internal_step(prow, crow, nn):
        m = nn * _BS
        hc = h_all[pl.ds(crow, 2 * m), :]
        u = jnp.tanh(_dot(hc, sw_v[:, :]) + sb_ref[:, :])
        s = jnp.tanh(_dot(u, cw_b))
        s4 = s.reshape(nn, 2, _BS, 1)
        w1 = jax.nn.sigmoid(s4[:, 0] - s4[:, 1])
        hc4 = hc.reshape(nn, 2, _BS, _E)
        h0 = (w1 * hc4[:, 0] + (1.0 - w1) * hc4[:, 1]).reshape(m, _E)
        gi = gi_at(prow, m)
        gh = _dot_t(h0, whh_v[:, :]) + bhh
        h = _gru_combine(gi, gh, h0)
        h_all[pl.ds(prow, m), :] = h
        return h

    def fold(acc, h, nn):
        m = jnp.max(h.reshape(nn, _BS, _E), axis=0) if nn > 1 else h
        return m if acc is None else jnp.maximum(acc, m)

    acc = None
    for k in range(4):
        xcp[k].wait()
        acc = fold(acc, leaf_step(3968 + k * 1024, 8), 8)
    icp.wait()
    wcp.wait()
    scp.wait()
    for half in range(2):
        acc = fold(acc, internal_step(1920 + half * 1024,
                                      3968 + half * 2048, 8), 8)
    acc = fold(acc, internal_step(896, 1920, 8), 8)
    acc = fold(acc, internal_step(384, 896, 4), 4)
    acc = fold(acc, internal_step(128, 384, 2), 2)
    acc = fold(acc, internal_step(0, 128, 1), 1)
    out_ref[:, :] = acc


def _tree_gru(x, wih, whh, bih, bhh, sw, sb, cw):
    return pl.pallas_call(
        _tree_body,
        in_specs=[pl.BlockSpec(memory_space=pltpu.HBM),
                  pl.BlockSpec(memory_space=pltpu.VMEM),
                  pl.BlockSpec(memory_space=pltpu.HBM),
                  pl.BlockSpec(memory_space=pltpu.VMEM),
                  pl.BlockSpec(memory_space=pltpu.VMEM),
                  pl.BlockSpec(memory_space=pltpu.HBM),
                  pl.BlockSpec(memory_space=pltpu.VMEM),
                  pl.BlockSpec(memory_space=pltpu.VMEM)],
        out_specs=pl.BlockSpec(memory_space=pltpu.VMEM),
        out_shape=jax.ShapeDtypeStruct((_BS, _E), jnp.float32),
        scratch_shapes=[
            pltpu.VMEM((_N * _BS, _E), jnp.float32),
            pltpu.VMEM((_N * _BS, _E), jnp.float32),
            pltpu.VMEM((3 * _E, _E), jnp.float32),
            pltpu.VMEM((_E, _E), jnp.float32),
            pltpu.SemaphoreType.DMA((4,)),
            pltpu.SemaphoreType.DMA,
            pltpu.SemaphoreType.DMA,
            pltpu.SemaphoreType.DMA,
        ],
    )(x, wih, whh, bih, bhh, sw, sb, cw)


def kernel(node_ids, emb, Wih, Whh, bih, bhh, sent_w, sent_b, ctx_w):
    x = _make_sc_gather()(emb, node_ids)
    return _tree_gru(x, Wih, Whh, bih, bhh, sent_w, sent_b, ctx_w)
